# baseline (speedup 1.0000x reference)
import functools

import jax
import jax.numpy as jnp
from jax import lax
from jax.experimental import pallas as pl
from jax.experimental.pallas import tpu as pltpu
from jax.experimental.pallas import tpu_sc as plsc

N = 10000
D = 128
C = 40
NP = 10240
DP = 48
EB = 128
NTILES = 32
NIT = 84
EPT = EB * NIT
EP = EPT * NTILES
RPT = NP // 16
RB = 128
NCH = RPT // RB
NBUF = 6

_mesh = plsc.VectorSubcoreMesh(core_axis_name="c", subcore_axis_name="s")
_sc_params = pltpu.CompilerParams(
    use_tc_tiling_on_sc=False, needs_layout_passes=False
)


def _zero_buf(buf, nrows, width):
    zero = jnp.zeros((16,), jnp.float32)

    def row(i, _):
        for j in range(width // 16):
            buf[i, pl.ds(j * 16, 16)] = zero
        return 0

    lax.fori_loop(0, nrows, row, 0)


def _rsqrt16(x):
    xi = plsc.bitcast(x, jnp.int32)
    yi = jnp.int32(0x5F3759DF) - (xi >> 1)
    y = plsc.bitcast(yi, jnp.float32)
    for _ in range(3):
        y = y * (1.5 - 0.5 * x * y * y)
    return y


def _hop_phase(table, accum, sidx, didx, ring, gsems, ssems):
    g = [
        pltpu.make_async_copy(
            table.at[sidx.at[j]], ring.at[j % NBUF], gsems[j % NBUF]
        )
        for j in range(NIT)
    ]
    sc = [
        pltpu.make_async_copy(
            ring.at[j % NBUF], accum.at[didx.at[j]], ssems[j % NBUF]
        )
        for j in range(NIT)
    ]
    LA = 3
    for j in range(LA):
        g[j].start()
    for j in range(NIT):
        nj = j + LA
        if nj < NIT:
            if nj >= NBUF:
                sc[nj - NBUF].wait()
            g[nj].start()
        g[j].wait()
        sc[j].start(add=True)
    for j in range(NIT - NBUF, NIT):
        sc[j].wait()


@functools.partial(
    pl.kernel,
    out_type=jax.ShapeDtypeStruct((2, NP, 16), jnp.float32),
    mesh=_mesh,
    compiler_params=_sc_params,
    scratch_types=[
        pltpu.VMEM((NIT, EB), jnp.int32),
        pltpu.VMEM((EB, 16), jnp.float32),
        pltpu.VMEM_SHARED((NP, 16), jnp.float32),
        pltpu.SemaphoreType.DMA,
    ],
)
def _deg_kernel(dst_hbm, out_hbm, didx, buf, accum, sem):
    c = lax.axis_index("c")
    s = lax.axis_index("s")
    w = c * 16 + s

    _zero_buf(buf, EB, 16)
    for j in range(NCH):
        pltpu.sync_copy(buf, accum.at[pl.ds(s * RPT + j * RB, RB)])

    pltpu.sync_copy(dst_hbm.at[w], didx)
    one = jnp.ones((16,), jnp.float32)

    def fill(i, _):
        buf[i, pl.ds(0, 16)] = one
        return 0

    lax.fori_loop(0, EB, fill, 0)
    plsc.subcore_barrier()

    descs = [
        pltpu.make_async_copy(buf, accum.at[didx.at[j]], sem)
        for j in range(NIT)
    ]
    for d in descs:
        d.start(add=True)
    for d in descs:
        d.wait()
    plsc.subcore_barrier()

    for j in range(NCH):
        r = s * RPT + j * RB
        pltpu.sync_copy(accum.at[pl.ds(r, RB)], buf)
        pltpu.sync_copy(buf, out_hbm.at[c, pl.ds(r, RB)])


_hop_scratch = [
    pltpu.VMEM((NIT, EB), jnp.int32),
    pltpu.VMEM((NIT, EB), jnp.int32),
    pltpu.VMEM((NBUF, EB, DP), jnp.float32),
    pltpu.VMEM((EB, 16), jnp.float32),
    pltpu.VMEM((EB, 16), jnp.float32),
    pltpu.VMEM_SHARED((NP, DP), jnp.float32),
    pltpu.VMEM_SHARED((NP, DP), jnp.float32),
    [pltpu.SemaphoreType.DMA] * NBUF,
    [pltpu.SemaphoreType.DMA] * NBUF,
]


@functools.partial(
    pl.kernel,
    out_type=jax.ShapeDtypeStruct((2, NP, DP), jnp.float32),
    mesh=_mesh,
    compiler_params=_sc_params,
    scratch_types=_hop_scratch,
)
def _hop1_kernel(y_hbm, deg_hbm, src_hbm, dst_hbm, p_out, sidx, didx, ring,
                 obuf, obuf2, table, accum, gsems, ssems):
    c = lax.axis_index("c")
    s = lax.axis_index("s")
    w = c * 16 + s

    _zero_buf(ring.at[0], RB, DP)
    for j in range(NCH):
        pltpu.sync_copy(ring.at[0], accum.at[pl.ds(s * RPT + j * RB, RB)])
    pltpu.sync_copy(src_hbm.at[w], sidx)
    pltpu.sync_copy(dst_hbm.at[w], didx)

    for j in range(NCH):
        r = s * RPT + j * RB
        pltpu.sync_copy(y_hbm.at[pl.ds(r, RB)], ring.at[1])
        pltpu.sync_copy(deg_hbm.at[0, pl.ds(r, RB)], obuf)
        pltpu.sync_copy(deg_hbm.at[1, pl.ds(r, RB)], obuf2)

        def srow(i, _):
            dv = _rsqrt16(obuf[i, pl.ds(0, 16)] + obuf2[i, pl.ds(0, 16)])
            for k in range(DP // 16):
                ring[1, i, pl.ds(k * 16, 16)] = (
                    ring[1, i, pl.ds(k * 16, 16)] * dv
                )
            return 0

        lax.fori_loop(0, RB, srow, 0)
        pltpu.sync_copy(ring.at[1], table.at[pl.ds(r, RB)])
    plsc.subcore_barrier()

    _hop_phase(table, accum, sidx, didx, ring, gsems, ssems)
    plsc.subcore_barrier()

    for j in range(NCH):
        r = s * RPT + j * RB
        pltpu.sync_copy(accum.at[pl.ds(r, RB)], ring.at[0])
        pltpu.sync_copy(ring.at[0], p_out.at[c, pl.ds(r, RB)])


@functools.partial(
    pl.kernel,
    out_type=jax.ShapeDtypeStruct((2, NP, DP), jnp.float32),
    mesh=_mesh,
    compiler_params=_sc_params,
    scratch_types=_hop_scratch,
)
def _hop2_kernel(p_hbm, deg_hbm, src_hbm, dst_hbm, q_out, sidx, didx, ring,
                 obuf, obuf2, table, accum, gsems, ssems):
    c = lax.axis_index("c")
    s = lax.axis_index("s")
    w = c * 16 + s

    _zero_buf(ring.at[0], RB, DP)
    for j in range(NCH):
        pltpu.sync_copy(ring.at[0], accum.at[pl.ds(s * RPT + j * RB, RB)])
    pltpu.sync_copy(src_hbm.at[w], sidx)
    pltpu.sync_copy(dst_hbm.at[w], didx)

    for j in range(NCH):
        r = s * RPT + j * RB
        pltpu.sync_copy(p_hbm.at[0, pl.ds(r, RB)], ring.at[1])
        pltpu.sync_copy(p_hbm.at[1, pl.ds(r, RB)], ring.at[2])
        pltpu.sync_copy(deg_hbm.at[0, pl.ds(r, RB)], obuf)
        pltpu.sync_copy(deg_hbm.at[1, pl.ds(r, RB)], obuf2)

        def srow(i, _):
            dv = _rsqrt16(obuf[i, pl.ds(0, 16)] + obuf2[i, pl.ds(0, 16)])
            dv2 = dv * dv
            for k in range(DP // 16):
                ring[1, i, pl.ds(k * 16, 16)] = (
                    ring[1, i, pl.ds(k * 16, 16)]
                    + ring[2, i, pl.ds(k * 16, 16)]
                ) * dv2
            return 0

        lax.fori_loop(0, RB, srow, 0)
        pltpu.sync_copy(ring.at[1], table.at[pl.ds(r, RB)])
    plsc.subcore_barrier()

    _hop_phase(table, accum, sidx, didx, ring, gsems, ssems)
    plsc.subcore_barrier()

    for j in range(NCH):
        r = s * RPT + j * RB
        pltpu.sync_copy(accum.at[pl.ds(r, RB)], ring.at[0])
        pltpu.sync_copy(deg_hbm.at[0, pl.ds(r, RB)], obuf)
        pltpu.sync_copy(deg_hbm.at[1, pl.ds(r, RB)], obuf2)

        def wrow(i, _):
            dv = _rsqrt16(obuf[i, pl.ds(0, 16)] + obuf2[i, pl.ds(0, 16)])
            for k in range(DP // 16):
                ring[0, i, pl.ds(k * 16, 16)] = (
                    ring[0, i, pl.ds(k * 16, 16)] * dv
                )
            return 0

        lax.fori_loop(0, RB, wrow, 0)
        pltpu.sync_copy(ring.at[0], q_out.at[c, pl.ds(r, RB)])


_BM = 2000
_GRID = N // _BM


def _proj_body(x_ref, wt_ref, y_ref):
    y_ref[...] = jnp.dot(
        x_ref[...], wt_ref[...], preferred_element_type=jnp.float32
    )


def _out_body(q_ref, b_ref, o_ref):
    logits = q_ref[0] + q_ref[1] + b_ref[...]
    m = jnp.max(logits, axis=1, keepdims=True)
    z = logits - m
    o_ref[...] = z - jnp.log(jnp.sum(jnp.exp(z), axis=1, keepdims=True))


_proj_call = pl.pallas_call(
    _proj_body,
    grid=(_GRID,),
    in_specs=[
        pl.BlockSpec((_BM, D), lambda i: (i, 0)),
        pl.BlockSpec((D, DP), lambda i: (0, 0)),
    ],
    out_specs=pl.BlockSpec((_BM, DP), lambda i: (i, 0)),
    out_shape=jax.ShapeDtypeStruct((NP, DP), jnp.float32),
)

_out_call = pl.pallas_call(
    _out_body,
    grid=(_GRID,),
    in_specs=[
        pl.BlockSpec((2, _BM, DP), lambda i: (0, i, 0)),
        pl.BlockSpec((1, DP), lambda i: (0, 0)),
    ],
    out_specs=pl.BlockSpec((_BM, DP), lambda i: (i, 0)),
    out_shape=jax.ShapeDtypeStruct((N, DP), jnp.float32),
)


def kernel(x, edge_index, W, b):
    src = edge_index[0].astype(jnp.int32)
    dst = edge_index[1].astype(jnp.int32)
    e = src.shape[0]
    loop = jnp.arange(N, dtype=jnp.int32)
    npad = EP - (e + N)
    trash = (jnp.arange(npad, dtype=jnp.int32) % (NP - N)) + N
    src_f = jnp.concatenate([src, loop, trash]).reshape(NTILES, NIT, EB)
    dst_f = jnp.concatenate([dst, loop, trash]).reshape(NTILES, NIT, EB)

    wt = jnp.zeros((D, DP), jnp.float32).at[:, :C].set(W.T)
    b48 = jnp.full((1, DP), -1e30, jnp.float32).at[0, :C].set(b)

    degp = _deg_kernel(dst_f)
    y = _proj_call(x, wt)
    p = _hop1_kernel(y, degp, src_f, dst_f)
    q = _hop2_kernel(p, degp, src_f, dst_f)
    out = _out_call(q, b48)
    return out[:N, :C]


# speedup vs baseline: 1.3101x; 1.3101x over previous
import functools

import jax
import jax.numpy as jnp
from jax import lax
from jax.experimental import pallas as pl
from jax.experimental.pallas import tpu as pltpu
from jax.experimental.pallas import tpu_sc as plsc

N = 10000
D = 128
C = 40
NP = 10240
DP = 48
EB = 128
NTILES = 32
NIT = 84
EPT = EB * NIT
EP = EPT * NTILES
RPT = NP // 16
RB = 128
NBUF = 6

_mesh = plsc.VectorSubcoreMesh(core_axis_name="c", subcore_axis_name="s")


def _zero_buf(buf, nrows, width):
    zero = jnp.zeros((16,), jnp.float32)

    def row(i, _):
        for j in range(width // 16):
            buf[i, pl.ds(j * 16, 16)] = zero
        return 0

    lax.fori_loop(0, nrows, row, 0)


@functools.partial(
    pl.kernel,
    out_type=jax.ShapeDtypeStruct((2, NP, 16), jnp.float32),
    mesh=_mesh,
    compiler_params=pltpu.CompilerParams(use_tc_tiling_on_sc=False),
    scratch_types=[
        pltpu.VMEM((NIT, EB), jnp.int32),
        pltpu.VMEM((EB, 16), jnp.float32),
        pltpu.VMEM_SHARED((NP, 16), jnp.float32),
        pltpu.SemaphoreType.DMA,
    ],
)
def _deg_kernel(ei_hbm, out_hbm, didx, buf, accum, sem):
    c = lax.axis_index("c")
    s = lax.axis_index("s")
    w = c * 16 + s

    _zero_buf(buf, EB, 16)
    for j in range(RPT // RB):
        pltpu.sync_copy(buf, accum.at[pl.ds(s * RPT + j * RB, RB)])

    pltpu.sync_copy(ei_hbm.at[1, w], didx)
    one = jnp.ones((16,), jnp.float32)

    def fill(i, _):
        buf[i, pl.ds(0, 16)] = one
        return 0

    lax.fori_loop(0, EB, fill, 0)
    plsc.subcore_barrier()

    descs = [
        pltpu.make_async_copy(buf, accum.at[didx.at[j]], sem)
        for j in range(NIT)
    ]
    for d in descs:
        d.start(add=True)
    for d in descs:
        d.wait()
    plsc.subcore_barrier()

    for j in range(RPT // RB):
        r = s * RPT + j * RB
        pltpu.sync_copy(accum.at[pl.ds(r, RB)], buf)
        pltpu.sync_copy(buf, out_hbm.at[c, pl.ds(r, RB)])


@functools.partial(
    pl.kernel,
    out_type=jax.ShapeDtypeStruct((2, NP, DP), jnp.float32),
    mesh=_mesh,
    compiler_params=pltpu.CompilerParams(use_tc_tiling_on_sc=False),
    scratch_types=[
        pltpu.VMEM((NIT, EB), jnp.int32),
        pltpu.VMEM((NIT, EB), jnp.int32),
        pltpu.VMEM((NBUF, EB, DP), jnp.float32),
        pltpu.VMEM_SHARED((NP, DP), jnp.float32),
        [pltpu.SemaphoreType.DMA] * NBUF,
        [pltpu.SemaphoreType.DMA] * NBUF,
    ],
)
def _hop_kernel(y_hbm, ei_hbm, out_hbm, sidx, didx, ring, accum,
                gsems, ssems):
    c = lax.axis_index("c")
    s = lax.axis_index("s")
    w = c * 16 + s

    pltpu.sync_copy(ei_hbm.at[0, w], sidx)
    pltpu.sync_copy(ei_hbm.at[1, w], didx)
    _zero_buf(ring.at[0], RB, DP)
    for j in range(RPT // RB):
        pltpu.sync_copy(ring.at[0], accum.at[pl.ds(s * RPT + j * RB, RB)])
    plsc.subcore_barrier()

    g = [
        pltpu.make_async_copy(
            y_hbm.at[sidx.at[j]], ring.at[j % NBUF], gsems[j % NBUF]
        )
        for j in range(NIT)
    ]
    sc = [
        pltpu.make_async_copy(
            ring.at[j % NBUF], accum.at[didx.at[j]], ssems[j % NBUF]
        )
        for j in range(NIT)
    ]
    LA = 3
    for j in range(LA):
        g[j].start()
    for j in range(NIT):
        nj = j + LA
        if nj < NIT:
            if nj >= NBUF:
                sc[nj - NBUF].wait()
            g[nj].start()
        g[j].wait()
        sc[j].start(add=True)
    for j in range(NIT - NBUF, NIT):
        sc[j].wait()
    plsc.subcore_barrier()

    for j in range(RPT // RB):
        r = s * RPT + j * RB
        pltpu.sync_copy(accum.at[pl.ds(r, RB)], ring.at[0])
        pltpu.sync_copy(ring.at[0], out_hbm.at[c, pl.ds(r, RB)])


_BM = 2048
_GRID = NP // _BM


def _proj_body(x_ref, wt_ref, d0_ref, d1_ref, y_ref, dis_ref):
    deg = d0_ref[...] + d1_ref[...]
    dis = jnp.where(deg > 0, lax.rsqrt(deg), 0.0)
    y = jnp.dot(x_ref[...], wt_ref[...], preferred_element_type=jnp.float32)
    y_ref[...] = y * dis
    dis_ref[...] = dis


def _mid_body(p_ref, dis_ref, t_ref):
    dis = dis_ref[...]
    t_ref[...] = (p_ref[0] + p_ref[1]) * (dis * dis)


def _out_body(q_ref, dis_ref, b_ref, o_ref):
    logits = (q_ref[0] + q_ref[1]) * dis_ref[...] + b_ref[...]
    m = jnp.max(logits, axis=1, keepdims=True)
    z = logits - m
    o_ref[...] = z - jnp.log(jnp.sum(jnp.exp(z), axis=1, keepdims=True))


_proj_call = pl.pallas_call(
    _proj_body,
    grid=(_GRID,),
    in_specs=[
        pl.BlockSpec((_BM, D), lambda i: (i, 0)),
        pl.BlockSpec((D, DP), lambda i: (0, 0)),
        pl.BlockSpec((_BM, 1), lambda i: (i, 0)),
        pl.BlockSpec((_BM, 1), lambda i: (i, 0)),
    ],
    out_specs=[
        pl.BlockSpec((_BM, DP), lambda i: (i, 0)),
        pl.BlockSpec((_BM, 1), lambda i: (i, 0)),
    ],
    out_shape=[
        jax.ShapeDtypeStruct((NP, DP), jnp.float32),
        jax.ShapeDtypeStruct((NP, 1), jnp.float32),
    ],
)

_mid_call = pl.pallas_call(
    _mid_body,
    grid=(_GRID,),
    in_specs=[
        pl.BlockSpec((2, _BM, DP), lambda i: (0, i, 0)),
        pl.BlockSpec((_BM, 1), lambda i: (i, 0)),
    ],
    out_specs=pl.BlockSpec((_BM, DP), lambda i: (i, 0)),
    out_shape=jax.ShapeDtypeStruct((NP, DP), jnp.float32),
)

_out_call = pl.pallas_call(
    _out_body,
    grid=(_GRID,),
    in_specs=[
        pl.BlockSpec((2, _BM, DP), lambda i: (0, i, 0)),
        pl.BlockSpec((_BM, 1), lambda i: (i, 0)),
        pl.BlockSpec((1, DP), lambda i: (0, 0)),
    ],
    out_specs=pl.BlockSpec((_BM, DP), lambda i: (i, 0)),
    out_shape=jax.ShapeDtypeStruct((NP, DP), jnp.float32),
)


def kernel(x, edge_index, W, b):
    e = edge_index.shape[1]
    loop = jnp.arange(N, dtype=jnp.int32)
    npad = EP - (e + N)
    trash = (jnp.arange(npad, dtype=jnp.int32) % (NP - N)) + N
    pad2 = jnp.stack(
        [jnp.concatenate([loop, trash]), jnp.concatenate([loop, trash])]
    )
    ei_f = jnp.concatenate(
        [edge_index.astype(jnp.int32), pad2], axis=1
    ).reshape(2, NTILES, NIT, EB)

    x_pad = jnp.zeros((NP, D), jnp.float32).at[:N].set(x)
    wt = jnp.zeros((D, DP), jnp.float32).at[:, :C].set(W.T)
    b48 = jnp.full((1, DP), -1e30, jnp.float32).at[0, :C].set(b)

    degp = _deg_kernel(ei_f)
    d0 = degp[0, :, 0:1]
    d1 = degp[1, :, 0:1]

    y, dis = _proj_call(x_pad, wt, d0, d1)
    p = _hop_kernel(y, ei_f)
    t = _mid_call(p, dis)
    q = _hop_kernel(t, ei_f)
    out = _out_call(q, dis, b48)
    return out[:N, :C]


# speedup vs baseline: 1.3141x; 1.0030x over previous
import functools

import jax
import jax.numpy as jnp
from jax import lax
from jax.experimental import pallas as pl
from jax.experimental.pallas import tpu as pltpu
from jax.experimental.pallas import tpu_sc as plsc

N = 10000
D = 128
C = 40
NP = 10240
DP = 48
EB = 128
NTILES = 32
NIT = 84
EPT = EB * NIT
EP = EPT * NTILES
RPT = NP // 16
RB = 128
NBUF = 6

_mesh = plsc.VectorSubcoreMesh(core_axis_name="c", subcore_axis_name="s")


def _zero_buf(buf, nrows, width):
    zero = jnp.zeros((16,), jnp.float32)

    def row(i, _):
        for j in range(width // 16):
            buf[i, pl.ds(j * 16, 16)] = zero
        return 0

    lax.fori_loop(0, nrows, row, 0)


@functools.partial(
    pl.kernel,
    out_type=jax.ShapeDtypeStruct((2, NP, 16), jnp.float32),
    mesh=_mesh,
    compiler_params=pltpu.CompilerParams(use_tc_tiling_on_sc=False),
    scratch_types=[
        pltpu.VMEM((NIT, EB), jnp.int32),
        pltpu.VMEM((EB, 16), jnp.float32),
        pltpu.VMEM_SHARED((NP, 16), jnp.float32),
        pltpu.SemaphoreType.DMA,
    ],
)
def _deg_kernel(ei_hbm, out_hbm, didx, buf, accum, sem):
    c = lax.axis_index("c")
    s = lax.axis_index("s")
    w = c * 16 + s

    _zero_buf(buf, EB, 16)
    for j in range(RPT // RB):
        pltpu.sync_copy(buf, accum.at[pl.ds(s * RPT + j * RB, RB)])

    pltpu.sync_copy(ei_hbm.at[1, w], didx)
    one = jnp.ones((16,), jnp.float32)

    def fill(i, _):
        buf[i, pl.ds(0, 16)] = one
        return 0

    lax.fori_loop(0, EB, fill, 0)
    plsc.subcore_barrier()

    descs = [
        pltpu.make_async_copy(buf, accum.at[didx.at[j]], sem)
        for j in range(NIT)
    ]
    for d in descs:
        d.start(add=True)
    for d in descs:
        d.wait()
    plsc.subcore_barrier()

    for j in range(RPT // RB):
        r = s * RPT + j * RB
        pltpu.sync_copy(accum.at[pl.ds(r, RB)], buf)
        pltpu.sync_copy(buf, out_hbm.at[c, pl.ds(r, RB)])


@functools.partial(
    pl.kernel,
    out_type=jax.ShapeDtypeStruct((2, NP, DP), jnp.float32),
    mesh=_mesh,
    compiler_params=pltpu.CompilerParams(use_tc_tiling_on_sc=False),
    scratch_types=[
        pltpu.VMEM((NIT, EB), jnp.int32),
        pltpu.VMEM((NIT, EB), jnp.int32),
        pltpu.VMEM((NBUF, EB, DP), jnp.float32),
        pltpu.VMEM_SHARED((NP, DP), jnp.float32),
        [pltpu.SemaphoreType.DMA] * NBUF,
        [pltpu.SemaphoreType.DMA] * NBUF,
    ],
)
def _hop_kernel(y_hbm, ei_hbm, out_hbm, sidx, didx, ring, accum,
                gsems, ssems):
    c = lax.axis_index("c")
    s = lax.axis_index("s")
    w = c * 16 + s

    pltpu.sync_copy(ei_hbm.at[0, w], sidx)
    pltpu.sync_copy(ei_hbm.at[1, w], didx)
    _zero_buf(ring.at[0], RB, DP)
    for j in range(RPT // RB):
        pltpu.sync_copy(ring.at[0], accum.at[pl.ds(s * RPT + j * RB, RB)])
    plsc.subcore_barrier()

    g = [
        pltpu.make_async_copy(
            y_hbm.at[sidx.at[j]], ring.at[j % NBUF], gsems[j % NBUF]
        )
        for j in range(NIT)
    ]
    sc = [
        pltpu.make_async_copy(
            ring.at[j % NBUF], accum.at[didx.at[j]], ssems[j % NBUF]
        )
        for j in range(NIT)
    ]
    LA = 3
    for j in range(LA):
        g[j].start()
    for j in range(NIT):
        nj = j + LA
        if nj < NIT:
            if nj >= NBUF:
                sc[nj - NBUF].wait()
            g[nj].start()
        g[j].wait()
        sc[j].start(add=True)
    for j in range(NIT - NBUF, NIT):
        sc[j].wait()
    plsc.subcore_barrier()

    for j in range(RPT // RB):
        r = s * RPT + j * RB
        pltpu.sync_copy(accum.at[pl.ds(r, RB)], ring.at[0])
        pltpu.sync_copy(ring.at[0], out_hbm.at[c, pl.ds(r, RB)])


_BM = 2048
_GRID = NP // _BM


def _proj_body(x_ref, wt_ref, d0_ref, d1_ref, y_ref, dis_ref):
    deg = d0_ref[...] + d1_ref[...]
    dis = jnp.where(deg > 0, lax.rsqrt(deg), 0.0)
    y = jnp.dot(x_ref[...], wt_ref[...], preferred_element_type=jnp.float32)
    y_ref[...] = y * dis
    dis_ref[...] = dis


def _mid_body(p_ref, dis_ref, t_ref):
    dis = dis_ref[...]
    t_ref[...] = (p_ref[0] + p_ref[1]) * (dis * dis)


def _out_body(q_ref, dis_ref, b_ref, o_ref):
    logits = (q_ref[0] + q_ref[1]) * dis_ref[...] + b_ref[...]
    m = jnp.max(logits, axis=1, keepdims=True)
    z = logits - m
    o_ref[...] = (z - jnp.log(jnp.sum(jnp.exp(z), axis=1, keepdims=True)))[
        :, :C
    ]


_proj_call = pl.pallas_call(
    _proj_body,
    grid=(_GRID,),
    in_specs=[
        pl.BlockSpec((_BM, D), lambda i: (i, 0)),
        pl.BlockSpec((D, DP), lambda i: (0, 0)),
        pl.BlockSpec((_BM, 1), lambda i: (i, 0)),
        pl.BlockSpec((_BM, 1), lambda i: (i, 0)),
    ],
    out_specs=[
        pl.BlockSpec((_BM, DP), lambda i: (i, 0)),
        pl.BlockSpec((_BM, 1), lambda i: (i, 0)),
    ],
    out_shape=[
        jax.ShapeDtypeStruct((NP, DP), jnp.float32),
        jax.ShapeDtypeStruct((NP, 1), jnp.float32),
    ],
)

_mid_call = pl.pallas_call(
    _mid_body,
    grid=(_GRID,),
    in_specs=[
        pl.BlockSpec((2, _BM, DP), lambda i: (0, i, 0)),
        pl.BlockSpec((_BM, 1), lambda i: (i, 0)),
    ],
    out_specs=pl.BlockSpec((_BM, DP), lambda i: (i, 0)),
    out_shape=jax.ShapeDtypeStruct((NP, DP), jnp.float32),
)

_OBM = 2000

_out_call = pl.pallas_call(
    _out_body,
    grid=(N // _OBM,),
    in_specs=[
        pl.BlockSpec((2, _OBM, DP), lambda i: (0, i, 0)),
        pl.BlockSpec((_OBM, 1), lambda i: (i, 0)),
        pl.BlockSpec((1, DP), lambda i: (0, 0)),
    ],
    out_specs=pl.BlockSpec((_OBM, C), lambda i: (i, 0)),
    out_shape=jax.ShapeDtypeStruct((N, C), jnp.float32),
)


def kernel(x, edge_index, W, b):
    e = edge_index.shape[1]
    loop = jnp.arange(N, dtype=jnp.int32)
    npad = EP - (e + N)
    trash = (jnp.arange(npad, dtype=jnp.int32) % (NP - N)) + N
    pad2 = jnp.stack(
        [jnp.concatenate([loop, trash]), jnp.concatenate([loop, trash])]
    )
    ei_f = jnp.concatenate(
        [edge_index.astype(jnp.int32), pad2], axis=1
    ).reshape(2, NTILES, NIT, EB)

    x_pad = jnp.zeros((NP, D), jnp.float32).at[:N].set(x)
    wt = jnp.zeros((D, DP), jnp.float32).at[:, :C].set(W.T)
    b48 = jnp.full((1, DP), -1e30, jnp.float32).at[0, :C].set(b)

    degp = _deg_kernel(ei_f)
    d0 = degp[0, :, 0:1]
    d1 = degp[1, :, 0:1]

    y, dis = _proj_call(x_pad, wt, d0, d1)
    p = _hop_kernel(y, ei_f)
    t = _mid_call(p, dis)
    q = _hop_kernel(t, ei_f)
    return _out_call(q, dis, b48)
